# baseline (speedup 1.0000x reference)
import jax
import jax.numpy as jnp
from jax import lax
from jax.experimental import pallas as pl
from jax.experimental.pallas import tpu as pltpu, tpu_sc as plsc

_B = 16384
_D = 128
_NC = 2
_NS = 16
_NW = _NC * _NS
_BPW = _B // _NW


_NCHUNK = 8
_CH = _BPW // _NCHUNK


def _gather_body(table_hbm, idx_hbm, out_hbm, idx_v, rows_v, isem, gsem, ssem):
    wid = lax.axis_index("s") * _NC + lax.axis_index("c")
    base = wid * _BPW
    idx_copies = [
        pltpu.async_copy(
            idx_hbm.at[pl.ds(base + j * _CH, _CH)],
            idx_v.at[pl.ds(j * _CH, _CH)],
            isem,
        )
        for j in range(_NCHUNK)
    ]
    gathers = []
    for j in range(_NCHUNK):
        idx_copies[j].wait()
        gathers.append(
            pltpu.async_copy(
                table_hbm.at[idx_v.at[pl.ds(j * _CH, _CH)]], rows_v.at[j], gsem
            )
        )
    stores = []
    for j in range(_NCHUNK):
        gathers[j].wait()
        stores.append(
            pltpu.async_copy(
                rows_v.at[j], out_hbm.at[pl.ds(base + j * _CH, _CH)], ssem
            )
        )
    for s in stores:
        s.wait()


_gather = pl.kernel(
    _gather_body,
    out_type=jax.ShapeDtypeStruct((_B, _D), jnp.float32),
    mesh=plsc.VectorSubcoreMesh(core_axis_name="c", subcore_axis_name="s"),
    scratch_types=[
        pltpu.VMEM((_BPW,), jnp.int32),
        pltpu.VMEM((_NCHUNK, _CH, _D), jnp.float32),
        pltpu.SemaphoreType.DMA,
        pltpu.SemaphoreType.DMA,
        pltpu.SemaphoreType.DMA,
    ],
)


def kernel(labels, table):
    return _gather(table, labels.astype(jnp.int32))


# speedup vs baseline: 1.0273x; 1.0273x over previous
import jax
import jax.numpy as jnp
from jax import lax
from jax.experimental import pallas as pl
from jax.experimental.pallas import tpu as pltpu, tpu_sc as plsc

_B = 16384
_D = 128
_NC = 2
_NS = 16
_NW = _NC * _NS
_BPW = _B // _NW
_C0 = 64
_C1 = _BPW - _C0


def _gather_body(table_hbm, idx_hbm, out_hbm, idx_v, rows_v, isem, gsem, ssem):
    wid = lax.axis_index("s") * _NC + lax.axis_index("c")
    base = wid * _BPW
    c0 = pltpu.async_copy(
        idx_hbm.at[pl.ds(base, _C0)], idx_v.at[pl.ds(0, _C0)], isem
    )
    c1 = pltpu.async_copy(
        idx_hbm.at[pl.ds(base + _C0, _C1)], idx_v.at[pl.ds(_C0, _C1)], isem
    )
    c0.wait()
    g0 = pltpu.async_copy(
        table_hbm.at[idx_v.at[pl.ds(0, _C0)]], rows_v.at[pl.ds(0, _C0)], gsem
    )
    c1.wait()
    g1 = pltpu.async_copy(
        table_hbm.at[idx_v.at[pl.ds(_C0, _C1)]], rows_v.at[pl.ds(_C0, _C1)], gsem
    )
    g0.wait()
    s0 = pltpu.async_copy(
        rows_v.at[pl.ds(0, _C0)], out_hbm.at[pl.ds(base, _C0)], ssem
    )
    g1.wait()
    s1 = pltpu.async_copy(
        rows_v.at[pl.ds(_C0, _C1)], out_hbm.at[pl.ds(base + _C0, _C1)], ssem
    )
    s0.wait()
    s1.wait()


_gather = pl.kernel(
    _gather_body,
    out_type=jax.ShapeDtypeStruct((_B, _D), jnp.float32),
    mesh=plsc.VectorSubcoreMesh(core_axis_name="c", subcore_axis_name="s"),
    scratch_types=[
        pltpu.VMEM((_BPW,), jnp.int32),
        pltpu.VMEM((_BPW, _D), jnp.float32),
        pltpu.SemaphoreType.DMA,
        pltpu.SemaphoreType.DMA,
        pltpu.SemaphoreType.DMA,
    ],
)


def kernel(labels, table):
    return _gather(table, labels.astype(jnp.int32))


# speedup vs baseline: 1.0310x; 1.0035x over previous
import jax
import jax.numpy as jnp
from jax import lax
from jax.experimental import pallas as pl
from jax.experimental.pallas import tpu as pltpu, tpu_sc as plsc

_B = 16384
_D = 128
_NC = 2
_NS = 16
_NW = _NC * _NS
_BPW = 496
_XTRA = 32
_XBASE = _BPW * _NW


def _gather_body(table_hbm, idx_hbm, out_hbm, idx_v, xidx_v, rows_v, xrows_v,
                 gsem, xgsem):
    c = lax.axis_index("c")
    s = lax.axis_index("s")
    base = (s * _NC + c) * _BPW
    pltpu.sync_copy(idx_hbm.at[pl.ds(base, _BPW)], idx_v)
    g = pltpu.async_copy(table_hbm.at[idx_v], rows_v, gsem)

    @pl.when(c == 0)
    def _extra():
        xbase = _XBASE + s * _XTRA
        pltpu.sync_copy(idx_hbm.at[pl.ds(xbase, _XTRA)], xidx_v)
        pltpu.async_copy(table_hbm.at[xidx_v], xrows_v, xgsem).wait()
        pltpu.sync_copy(xrows_v, out_hbm.at[pl.ds(xbase, _XTRA)])

    g.wait()
    pltpu.sync_copy(rows_v, out_hbm.at[pl.ds(base, _BPW)])


_gather = pl.kernel(
    _gather_body,
    out_type=jax.ShapeDtypeStruct((_B, _D), jnp.float32),
    mesh=plsc.VectorSubcoreMesh(core_axis_name="c", subcore_axis_name="s"),
    scratch_types=[
        pltpu.VMEM((_BPW,), jnp.int32),
        pltpu.VMEM((_XTRA,), jnp.int32),
        pltpu.VMEM((_BPW, _D), jnp.float32),
        pltpu.VMEM((_XTRA, _D), jnp.float32),
        pltpu.SemaphoreType.DMA,
        pltpu.SemaphoreType.DMA,
    ],
)


def kernel(labels, table):
    return _gather(table, labels.astype(jnp.int32))


# speedup vs baseline: 1.0317x; 1.0007x over previous
import jax
import jax.numpy as jnp
from jax import lax
from jax.experimental import pallas as pl
from jax.experimental.pallas import tpu as pltpu, tpu_sc as plsc

_B = 16384
_D = 128
_NC = 2
_NS = 16
_NW = _NC * _NS
_BPW = _B // _NW


def _gather_body(table_hbm, idx_hbm, out_hbm, idx_v, rows_v, sem):
    wid = lax.axis_index("s") * _NC + lax.axis_index("c")
    base = wid * _BPW
    pltpu.sync_copy(idx_hbm.at[pl.ds(base, _BPW)], idx_v)
    pltpu.async_copy(table_hbm.at[idx_v], rows_v, sem).wait()
    pltpu.sync_copy(rows_v, out_hbm.at[pl.ds(base, _BPW)])


_gather = pl.kernel(
    _gather_body,
    out_type=jax.ShapeDtypeStruct((_B, _D), jnp.float32),
    mesh=plsc.VectorSubcoreMesh(core_axis_name="c", subcore_axis_name="s"),
    scratch_types=[
        pltpu.VMEM((_BPW,), jnp.int32),
        pltpu.VMEM((_BPW, _D), jnp.float32),
        pltpu.SemaphoreType.DMA,
    ],
)


def kernel(labels, table):
    return _gather(table, labels.astype(jnp.int32))
